# baseline (speedup 1.0000x reference)
import functools

import jax
import jax.numpy as jnp
from jax import lax
from jax.experimental import pallas as pl
from jax.experimental.pallas import tpu as pltpu
from jax.experimental.pallas import tpu_sc as plsc

NUM_EXPERTS = 64
TOP_K = 2
BATCH = 4
SEQ = 8192
TOKENS = BATCH * SEQ

_INFO = plsc.get_sparse_core_info()
NC = _INFO.num_cores
NS = _INFO.num_subcores
LANES = _INFO.num_lanes
NW = NC * NS
TPW = TOKENS // NW
GROUPS = TPW // LANES

_mesh = plsc.VectorSubcoreMesh(core_axis_name="c", subcore_axis_name="s")


@functools.partial(
    pl.kernel,
    out_type=(
        jax.ShapeDtypeStruct((TOKENS * TOP_K,), jnp.int32),
        jax.ShapeDtypeStruct((TOKENS * TOP_K,), jnp.float32),
        jax.ShapeDtypeStruct((TOKENS * TOP_K,), jnp.float32),
    ),
    mesh=_mesh,
    compiler_params=pltpu.CompilerParams(needs_layout_passes=False),
    scratch_types=[
        pltpu.VMEM((TPW * NUM_EXPERTS,), jnp.float32),
        pltpu.VMEM((NUM_EXPERTS,), jnp.float32),
        pltpu.VMEM((TPW * TOP_K,), jnp.int32),
        pltpu.VMEM((TPW * TOP_K,), jnp.float32),
        pltpu.VMEM((TPW * TOP_K,), jnp.float32),
    ],
)
def _auction(conf_hbm, wealth_hbm, oidx_hbm, orw_hbm, opay_hbm,
             conf_v, wealth_v, oidx_v, orw_v, opay_v):
    wid = lax.axis_index("s") * NC + lax.axis_index("c")
    base = wid * TPW
    pltpu.sync_copy(conf_hbm.at[pl.ds(base * NUM_EXPERTS, TPW * NUM_EXPERTS)],
                    conf_v)
    pltpu.sync_copy(wealth_hbm, wealth_v)

    iota = lax.iota(jnp.int32, LANES)
    zeros = jnp.zeros((LANES,), jnp.int32)
    neg_inf = jnp.full((LANES,), -jnp.inf, jnp.float32)

    def group_body(g, carry):
        tokv = iota + g * LANES
        idx0 = tokv * NUM_EXPERTS

        def expert_body(_, st):
            m1, m2, m3, i1, i2, ev, idxv = st
            col = plsc.load_gather(conf_v, [idxv])
            w = plsc.load_gather(wealth_v, [ev])
            b = col * w
            gt1 = b > m1
            gt2 = b > m2
            gt3 = b > m3
            nm3 = jnp.where(gt2, m2, jnp.where(gt3, b, m3))
            nm2 = jnp.where(gt1, m1, jnp.where(gt2, b, m2))
            ni2 = jnp.where(gt1, i1, jnp.where(gt2, ev, i2))
            nm1 = jnp.where(gt1, b, m1)
            ni1 = jnp.where(gt1, ev, i1)
            return nm1, nm2, nm3, ni1, ni2, ev + 1, idxv + 1

        m1, m2, m3, i1, i2, _, _ = lax.fori_loop(
            0, NUM_EXPERTS, expert_body,
            (neg_inf, neg_inf, neg_inf, zeros, zeros, zeros, idx0))

        t = jnp.exp(m2 - m1)
        inv = 1.0 / (1.0 + t)
        pos = tokv * TOP_K
        plsc.store_scatter(oidx_v, [pos], i1)
        plsc.store_scatter(oidx_v, [pos + 1], i2)
        plsc.store_scatter(orw_v, [pos], inv)
        plsc.store_scatter(orw_v, [pos + 1], t * inv)
        plsc.store_scatter(opay_v, [pos], m3)
        plsc.store_scatter(opay_v, [pos + 1], m3)
        return carry

    lax.fori_loop(0, GROUPS, group_body, 0)

    pltpu.sync_copy(oidx_v, oidx_hbm.at[pl.ds(base * TOP_K, TPW * TOP_K)])
    pltpu.sync_copy(orw_v, orw_hbm.at[pl.ds(base * TOP_K, TPW * TOP_K)])
    pltpu.sync_copy(opay_v, opay_hbm.at[pl.ds(base * TOP_K, TPW * TOP_K)])


def kernel(confidences, wealth):
    conf = confidences.reshape(TOKENS * NUM_EXPERTS)
    oidx, orw, opay = _auction(conf, wealth)
    shape = (BATCH, SEQ, TOP_K)
    return (oidx.reshape(shape), orw.reshape(shape), opay.reshape(shape))


# speedup vs baseline: 1.3258x; 1.3258x over previous
import functools

import jax
import jax.numpy as jnp
from jax import lax
from jax.experimental import pallas as pl
from jax.experimental.pallas import tpu as pltpu
from jax.experimental.pallas import tpu_sc as plsc

NUM_EXPERTS = 64
TOP_K = 2
BATCH = 4
SEQ = 8192
TOKENS = BATCH * SEQ

_INFO = plsc.get_sparse_core_info()
NC = _INFO.num_cores
NS = _INFO.num_subcores
LANES = _INFO.num_lanes
NW = NC * NS
TPW = TOKENS // NW
WPB = SEQ // TPW
CHUNK = 512
SUB = 128
GPS = SUB // LANES

_mesh = plsc.VectorSubcoreMesh(core_axis_name="c", subcore_axis_name="s")


@functools.partial(
    pl.kernel,
    out_type=(
        jax.ShapeDtypeStruct((BATCH, SEQ, TOP_K), jnp.int32),
        jax.ShapeDtypeStruct((BATCH, SEQ, TOP_K), jnp.float32),
        jax.ShapeDtypeStruct((BATCH, SEQ, TOP_K), jnp.float32),
    ),
    mesh=_mesh,
    compiler_params=pltpu.CompilerParams(needs_layout_passes=False),
    scratch_types=[
        pltpu.VMEM((CHUNK, NUM_EXPERTS), jnp.float32),
        pltpu.VMEM((NUM_EXPERTS,), jnp.float32),
        pltpu.VMEM((SUB, TOP_K), jnp.int32),
        pltpu.VMEM((SUB, TOP_K), jnp.float32),
        pltpu.VMEM((SUB, TOP_K), jnp.float32),
    ],
)
def _auction(conf_hbm, wealth_hbm, oidx_hbm, orw_hbm, opay_hbm,
             conf_v, wealth_v, oidx_v, orw_v, opay_v):
    wid = lax.axis_index("s") * NC + lax.axis_index("c")
    b = wid // WPB
    row0 = (wid % WPB) * TPW
    pltpu.sync_copy(wealth_hbm, wealth_v)

    iota = lax.iota(jnp.int32, LANES)
    zeros = jnp.zeros((LANES,), jnp.int32)
    ones = zeros + 1
    neg_inf = jnp.full((LANES,), -jnp.inf, jnp.float32)

    for chunk in range(TPW // CHUNK):
        pltpu.sync_copy(conf_hbm.at[b, pl.ds(row0 + chunk * CHUNK, CHUNK)],
                        conf_v)
        for sub in range(CHUNK // SUB):

            def group_body(g, carry):
                uloc = iota + g * LANES
                uv = uloc + sub * SUB

                def expert_body(_, st):
                    m1, m2, m3, i1, i2, ev = st
                    col = plsc.load_gather(conf_v, [uv, ev])
                    w = plsc.load_gather(wealth_v, [ev])
                    b_ = col * w
                    gt1 = b_ > m1
                    gt2 = b_ > m2
                    gt3 = b_ > m3
                    nm3 = jnp.where(gt2, m2, jnp.where(gt3, b_, m3))
                    nm2 = jnp.where(gt1, m1, jnp.where(gt2, b_, m2))
                    ni2 = jnp.where(gt1, i1, jnp.where(gt2, ev, i2))
                    nm1 = jnp.where(gt1, b_, m1)
                    ni1 = jnp.where(gt1, ev, i1)
                    return nm1, nm2, nm3, ni1, ni2, ev + 1

                m1, m2, m3, i1, i2, _ = lax.fori_loop(
                    0, NUM_EXPERTS, expert_body,
                    (neg_inf, neg_inf, neg_inf, zeros, zeros, zeros))

                t = jnp.exp(m2 - m1)
                inv = 1.0 / (1.0 + t)
                plsc.store_scatter(oidx_v, [uloc, zeros], i1)
                plsc.store_scatter(oidx_v, [uloc, ones], i2)
                plsc.store_scatter(orw_v, [uloc, zeros], inv)
                plsc.store_scatter(orw_v, [uloc, ones], t * inv)
                plsc.store_scatter(opay_v, [uloc, zeros], m3)
                plsc.store_scatter(opay_v, [uloc, ones], m3)
                return carry

            lax.fori_loop(0, GPS, group_body, 0)

            s0 = row0 + chunk * CHUNK + sub * SUB
            pltpu.sync_copy(oidx_v, oidx_hbm.at[b, pl.ds(s0, SUB)])
            pltpu.sync_copy(orw_v, orw_hbm.at[b, pl.ds(s0, SUB)])
            pltpu.sync_copy(opay_v, opay_hbm.at[b, pl.ds(s0, SUB)])


def kernel(confidences, wealth):
    return _auction(confidences, wealth)


# speedup vs baseline: 1.8963x; 1.4303x over previous
import functools

import jax
import jax.numpy as jnp
from jax import lax
from jax.experimental import pallas as pl
from jax.experimental.pallas import tpu as pltpu
from jax.experimental.pallas import tpu_sc as plsc

NUM_EXPERTS = 64
TOP_K = 2
BATCH = 4
SEQ = 8192
TOKENS = BATCH * SEQ

_INFO = plsc.get_sparse_core_info()
NC = _INFO.num_cores
NS = _INFO.num_subcores
LANES = _INFO.num_lanes
NW = NC * NS
TPW = TOKENS // NW
WPB = SEQ // TPW
CHUNK = 512
GPC = CHUNK // LANES

_mesh = plsc.VectorSubcoreMesh(core_axis_name="c", subcore_axis_name="s")


@functools.partial(
    pl.kernel,
    out_type=(
        jax.ShapeDtypeStruct((TOP_K, BATCH, SEQ), jnp.int32),
        jax.ShapeDtypeStruct((TOP_K, BATCH, SEQ), jnp.float32),
        jax.ShapeDtypeStruct((TOP_K, BATCH, SEQ), jnp.float32),
    ),
    mesh=_mesh,
    compiler_params=pltpu.CompilerParams(needs_layout_passes=False),
    scratch_types=[
        pltpu.VMEM((CHUNK, NUM_EXPERTS), jnp.float32),
        pltpu.VMEM((NUM_EXPERTS,), jnp.float32),
        pltpu.VMEM((TPW,), jnp.int32),
        pltpu.VMEM((TPW,), jnp.int32),
        pltpu.VMEM((TPW,), jnp.float32),
        pltpu.VMEM((TPW,), jnp.float32),
        pltpu.VMEM((TPW,), jnp.float32),
    ],
)
def _auction(conf_hbm, wealth_hbm, oidx_hbm, orw_hbm, opay_hbm,
             conf_v, wealth_v, e0_v, e1_v, w0_v, w1_v, p_v):
    wid = lax.axis_index("s") * NC + lax.axis_index("c")
    b = wid // WPB
    row0 = (wid % WPB) * TPW
    pltpu.sync_copy(wealth_hbm, wealth_v)

    iota = lax.iota(jnp.int32, LANES)
    zeros = jnp.zeros((LANES,), jnp.int32)
    neg_inf = jnp.full((LANES,), -jnp.inf, jnp.float32)

    for chunk in range(TPW // CHUNK):
        pltpu.sync_copy(conf_hbm.at[b, pl.ds(row0 + chunk * CHUNK, CHUNK)],
                        conf_v)

        def group_body(g, carry):
            uv = iota + g * LANES

            def expert_body(_, st):
                m1, m2, m3, i1, i2, ev = st
                col = plsc.load_gather(conf_v, [uv, ev])
                w = plsc.load_gather(wealth_v, [ev])
                b_ = col * w
                gt1 = b_ > m1
                gt2 = b_ > m2
                gt3 = b_ > m3
                nm3 = jnp.where(gt2, m2, jnp.where(gt3, b_, m3))
                nm2 = jnp.where(gt1, m1, jnp.where(gt2, b_, m2))
                ni2 = jnp.where(gt1, i1, jnp.where(gt2, ev, i2))
                nm1 = jnp.where(gt1, b_, m1)
                ni1 = jnp.where(gt1, ev, i1)
                return nm1, nm2, nm3, ni1, ni2, ev + 1

            m1, m2, m3, i1, i2, _ = lax.fori_loop(
                0, NUM_EXPERTS, expert_body,
                (neg_inf, neg_inf, neg_inf, zeros, zeros, zeros))

            t = jnp.exp(m2 - m1)
            inv = 1.0 / (1.0 + t)
            off = chunk * CHUNK + g * LANES
            e0_v[pl.ds(off, LANES)] = i1
            e1_v[pl.ds(off, LANES)] = i2
            w0_v[pl.ds(off, LANES)] = inv
            w1_v[pl.ds(off, LANES)] = t * inv
            p_v[pl.ds(off, LANES)] = m3
            return carry

        lax.fori_loop(0, GPC, group_body, 0)

    pltpu.sync_copy(e0_v, oidx_hbm.at[0, b, pl.ds(row0, TPW)])
    pltpu.sync_copy(e1_v, oidx_hbm.at[1, b, pl.ds(row0, TPW)])
    pltpu.sync_copy(w0_v, orw_hbm.at[0, b, pl.ds(row0, TPW)])
    pltpu.sync_copy(w1_v, orw_hbm.at[1, b, pl.ds(row0, TPW)])
    pltpu.sync_copy(p_v, opay_hbm.at[0, b, pl.ds(row0, TPW)])
    pltpu.sync_copy(p_v, opay_hbm.at[1, b, pl.ds(row0, TPW)])


def kernel(confidences, wealth):
    oidx, orw, opay = _auction(confidences, wealth)
    perm = (1, 2, 0)
    return (jnp.transpose(oidx, perm), jnp.transpose(orw, perm),
            jnp.transpose(opay, perm))


# speedup vs baseline: 2.0314x; 1.0712x over previous
import functools

import jax
import jax.numpy as jnp
from jax import lax
from jax.experimental import pallas as pl
from jax.experimental.pallas import tpu as pltpu
from jax.experimental.pallas import tpu_sc as plsc

NUM_EXPERTS = 64
TOP_K = 2
BATCH = 4
SEQ = 8192
TOKENS = BATCH * SEQ

_INFO = plsc.get_sparse_core_info()
NC = _INFO.num_cores
NS = _INFO.num_subcores
LANES = _INFO.num_lanes
NW = NC * NS
TPW = TOKENS // NW
WPB = SEQ // TPW
CHUNK = 512
GPC = CHUNK // LANES
UNROLL = 4

_mesh = plsc.VectorSubcoreMesh(core_axis_name="c", subcore_axis_name="s")


@functools.partial(
    pl.kernel,
    out_type=(
        jax.ShapeDtypeStruct((TOP_K, BATCH, SEQ), jnp.int32),
        jax.ShapeDtypeStruct((TOP_K, BATCH, SEQ), jnp.float32),
        jax.ShapeDtypeStruct((TOP_K, BATCH, SEQ), jnp.float32),
    ),
    mesh=_mesh,
    compiler_params=pltpu.CompilerParams(needs_layout_passes=False),
    scratch_types=[
        pltpu.VMEM((CHUNK, NUM_EXPERTS), jnp.float32),
        pltpu.VMEM((NUM_EXPERTS,), jnp.float32),
        pltpu.VMEM((TPW,), jnp.int32),
        pltpu.VMEM((TPW,), jnp.int32),
        pltpu.VMEM((TPW,), jnp.float32),
        pltpu.VMEM((TPW,), jnp.float32),
        pltpu.VMEM((TPW,), jnp.float32),
    ],
)
def _auction(conf_hbm, wealth_hbm, oidx_hbm, orw_hbm, opay_hbm,
             conf_v, wealth_v, e0_v, e1_v, w0_v, w1_v, p_v):
    wid = lax.axis_index("s") * NC + lax.axis_index("c")
    b = wid // WPB
    row0 = (wid % WPB) * TPW
    pltpu.sync_copy(wealth_hbm, wealth_v)

    iota = lax.iota(jnp.int32, LANES)
    zeros = jnp.zeros((LANES,), jnp.int32)
    neg_inf = jnp.full((LANES,), -jnp.inf, jnp.float32)

    for chunk in range(TPW // CHUNK):
        pltpu.sync_copy(conf_hbm.at[b, pl.ds(row0 + chunk * CHUNK, CHUNK)],
                        conf_v)

        def group_body(g, carry):
            uv = iota + g * LANES

            def expert_body(_, st):
                m1, m2, m3, i1, i2, ev = st
                for k in range(UNROLL):
                    evk = ev + k
                    col = plsc.load_gather(conf_v, [uv, evk])
                    w = plsc.load_gather(wealth_v, [evk])
                    b_ = col * w
                    gt1 = b_ > m1
                    gt2 = b_ > m2
                    gt3 = b_ > m3
                    nm3 = jnp.where(gt2, m2, jnp.where(gt3, b_, m3))
                    nm2 = jnp.where(gt1, m1, jnp.where(gt2, b_, m2))
                    ni2 = jnp.where(gt1, i1, jnp.where(gt2, evk, i2))
                    nm1 = jnp.where(gt1, b_, m1)
                    ni1 = jnp.where(gt1, evk, i1)
                    m1, m2, m3, i1, i2 = nm1, nm2, nm3, ni1, ni2
                return m1, m2, m3, i1, i2, ev + UNROLL

            m1, m2, m3, i1, i2, _ = lax.fori_loop(
                0, NUM_EXPERTS // UNROLL, expert_body,
                (neg_inf, neg_inf, neg_inf, zeros, zeros, zeros))

            t = jnp.exp(m2 - m1)
            inv = 1.0 / (1.0 + t)
            off = chunk * CHUNK + g * LANES
            e0_v[pl.ds(off, LANES)] = i1
            e1_v[pl.ds(off, LANES)] = i2
            w0_v[pl.ds(off, LANES)] = inv
            w1_v[pl.ds(off, LANES)] = t * inv
            p_v[pl.ds(off, LANES)] = m3
            return carry

        lax.fori_loop(0, GPC, group_body, 0)

    pltpu.sync_copy(e0_v, oidx_hbm.at[0, b, pl.ds(row0, TPW)])
    pltpu.sync_copy(e1_v, oidx_hbm.at[1, b, pl.ds(row0, TPW)])
    pltpu.sync_copy(w0_v, orw_hbm.at[0, b, pl.ds(row0, TPW)])
    pltpu.sync_copy(w1_v, orw_hbm.at[1, b, pl.ds(row0, TPW)])
    pltpu.sync_copy(p_v, opay_hbm.at[0, b, pl.ds(row0, TPW)])
    pltpu.sync_copy(p_v, opay_hbm.at[1, b, pl.ds(row0, TPW)])


def kernel(confidences, wealth):
    oidx, orw, opay = _auction(confidences, wealth)
    perm = (1, 2, 0)
    return (jnp.transpose(oidx, perm), jnp.transpose(orw, perm),
            jnp.transpose(opay, perm))


# speedup vs baseline: 2.1687x; 1.0676x over previous
import functools

import jax
import jax.numpy as jnp
from jax import lax
from jax.experimental import pallas as pl
from jax.experimental.pallas import tpu as pltpu
from jax.experimental.pallas import tpu_sc as plsc

NUM_EXPERTS = 64
TOP_K = 2
BATCH = 4
SEQ = 8192
TOKENS = BATCH * SEQ

_INFO = plsc.get_sparse_core_info()
NC = _INFO.num_cores
NS = _INFO.num_subcores
LANES = _INFO.num_lanes
NW = NC * NS
TPW = TOKENS // NW
WPB = SEQ // TPW
CHUNK = 512
GPC = CHUNK // LANES
UNROLL = 8

_mesh = plsc.VectorSubcoreMesh(core_axis_name="c", subcore_axis_name="s")


@functools.partial(
    pl.kernel,
    out_type=(
        jax.ShapeDtypeStruct((TOP_K, BATCH, SEQ), jnp.int32),
        jax.ShapeDtypeStruct((TOP_K, BATCH, SEQ), jnp.float32),
        jax.ShapeDtypeStruct((TOP_K, BATCH, SEQ), jnp.float32),
    ),
    mesh=_mesh,
    compiler_params=pltpu.CompilerParams(needs_layout_passes=False),
    scratch_types=[
        pltpu.VMEM((CHUNK, NUM_EXPERTS), jnp.float32),
        pltpu.VMEM((NUM_EXPERTS,), jnp.float32),
        pltpu.VMEM((TPW,), jnp.int32),
        pltpu.VMEM((TPW,), jnp.int32),
        pltpu.VMEM((TPW,), jnp.float32),
        pltpu.VMEM((TPW,), jnp.float32),
        pltpu.VMEM((TPW,), jnp.float32),
    ],
)
def _auction(conf_hbm, wealth_hbm, oidx_hbm, orw_hbm, opay_hbm,
             conf_v, wealth_v, e0_v, e1_v, w0_v, w1_v, p_v):
    wid = lax.axis_index("s") * NC + lax.axis_index("c")
    b = wid // WPB
    row0 = (wid % WPB) * TPW
    pltpu.sync_copy(wealth_hbm, wealth_v)

    iota = lax.iota(jnp.int32, LANES)
    zeros = jnp.zeros((LANES,), jnp.int32)
    neg_inf = jnp.full((LANES,), -jnp.inf, jnp.float32)

    for chunk in range(TPW // CHUNK):
        pltpu.sync_copy(conf_hbm.at[b, pl.ds(row0 + chunk * CHUNK, CHUNK)],
                        conf_v)

        def group_body(g, carry):
            uv = iota + g * LANES

            def expert_body(_, st):
                m1, m2, m3, i1, i2, ev = st
                for k in range(UNROLL):
                    evk = ev + k
                    col = plsc.load_gather(conf_v, [uv, evk])
                    w = plsc.load_gather(wealth_v, [evk])
                    b_ = col * w
                    gt1 = b_ > m1
                    gt2 = b_ > m2
                    nm3 = jnp.maximum(m3, jnp.minimum(m2, b_))
                    nm2 = jnp.maximum(m2, jnp.minimum(m1, b_))
                    ni2 = jnp.where(gt1, i1, jnp.where(gt2, evk, i2))
                    nm1 = jnp.maximum(m1, b_)
                    ni1 = jnp.where(gt1, evk, i1)
                    m1, m2, m3, i1, i2 = nm1, nm2, nm3, ni1, ni2
                return m1, m2, m3, i1, i2, ev + UNROLL

            m1, m2, m3, i1, i2, _ = lax.fori_loop(
                0, NUM_EXPERTS // UNROLL, expert_body,
                (neg_inf, neg_inf, neg_inf, zeros, zeros, zeros))

            t = jnp.exp(m2 - m1)
            inv = 1.0 / (1.0 + t)
            off = chunk * CHUNK + g * LANES
            e0_v[pl.ds(off, LANES)] = i1
            e1_v[pl.ds(off, LANES)] = i2
            w0_v[pl.ds(off, LANES)] = inv
            w1_v[pl.ds(off, LANES)] = t * inv
            p_v[pl.ds(off, LANES)] = m3
            return carry

        lax.fori_loop(0, GPC, group_body, 0)

    pltpu.sync_copy(e0_v, oidx_hbm.at[0, b, pl.ds(row0, TPW)])
    pltpu.sync_copy(e1_v, oidx_hbm.at[1, b, pl.ds(row0, TPW)])
    pltpu.sync_copy(w0_v, orw_hbm.at[0, b, pl.ds(row0, TPW)])
    pltpu.sync_copy(w1_v, orw_hbm.at[1, b, pl.ds(row0, TPW)])
    pltpu.sync_copy(p_v, opay_hbm.at[0, b, pl.ds(row0, TPW)])
    pltpu.sync_copy(p_v, opay_hbm.at[1, b, pl.ds(row0, TPW)])


def kernel(confidences, wealth):
    oidx, orw, opay = _auction(confidences, wealth)
    perm = (1, 2, 0)
    return (jnp.transpose(oidx, perm), jnp.transpose(orw, perm),
            jnp.transpose(opay, perm))


# speedup vs baseline: 2.2348x; 1.0305x over previous
import functools

import jax
import jax.numpy as jnp
from jax import lax
from jax.experimental import pallas as pl
from jax.experimental.pallas import tpu as pltpu
from jax.experimental.pallas import tpu_sc as plsc

NUM_EXPERTS = 64
TOP_K = 2
BATCH = 4
SEQ = 8192
TOKENS = BATCH * SEQ

_INFO = plsc.get_sparse_core_info()
NC = _INFO.num_cores
NS = _INFO.num_subcores
LANES = _INFO.num_lanes
NW = NC * NS
TPW = TOKENS // NW
WPB = SEQ // TPW
CHUNK = 256
GPC = CHUNK // LANES
UNROLL = 8

_mesh = plsc.VectorSubcoreMesh(core_axis_name="c", subcore_axis_name="s")


@functools.partial(
    pl.kernel,
    out_type=(
        jax.ShapeDtypeStruct((TOP_K, BATCH, SEQ), jnp.int32),
        jax.ShapeDtypeStruct((TOP_K, BATCH, SEQ), jnp.float32),
        jax.ShapeDtypeStruct((TOP_K, BATCH, SEQ), jnp.float32),
    ),
    mesh=_mesh,
    compiler_params=pltpu.CompilerParams(needs_layout_passes=False),
    scratch_types=[
        pltpu.VMEM((CHUNK, NUM_EXPERTS), jnp.float32),
        pltpu.VMEM((CHUNK, NUM_EXPERTS), jnp.float32),
        pltpu.SemaphoreType.DMA,
        pltpu.SemaphoreType.DMA,
        pltpu.VMEM((NUM_EXPERTS,), jnp.float32),
        pltpu.VMEM((TPW,), jnp.int32),
        pltpu.VMEM((TPW,), jnp.int32),
        pltpu.VMEM((TPW,), jnp.float32),
        pltpu.VMEM((TPW,), jnp.float32),
        pltpu.VMEM((TPW,), jnp.float32),
    ],
)
def _auction(conf_hbm, wealth_hbm, oidx_hbm, orw_hbm, opay_hbm,
             conf_a, conf_b, sem_a, sem_b, wealth_v, e0_v, e1_v, w0_v,
             w1_v, p_v):
    wid = lax.axis_index("s") * NC + lax.axis_index("c")
    b = wid // WPB
    row0 = (wid % WPB) * TPW
    pltpu.sync_copy(wealth_hbm, wealth_v)

    iota = lax.iota(jnp.int32, LANES)
    zeros = jnp.zeros((LANES,), jnp.int32)
    neg_inf = jnp.full((LANES,), -jnp.inf, jnp.float32)

    bufs = (conf_a, conf_b)
    sems = (sem_a, sem_b)
    nchunk = TPW // CHUNK

    def start_fetch(c):
        return pltpu.async_copy(
            conf_hbm.at[b, pl.ds(row0 + c * CHUNK, CHUNK)],
            bufs[c % 2], sems[c % 2])

    handles = [start_fetch(0), None]
    for chunk in range(nchunk):
        if chunk + 1 < nchunk:
            handles[(chunk + 1) % 2] = start_fetch(chunk + 1)
        handles[chunk % 2].wait()
        conf_v = bufs[chunk % 2]

        def group_body(g, carry):
            uv = iota + g * LANES

            def expert_body(_, st):
                m1, m2, m3, i1, i2, ev = st
                for k in range(UNROLL):
                    evk = ev + k
                    col = plsc.load_gather(conf_v, [uv, evk])
                    w = plsc.load_gather(wealth_v, [evk])
                    b_ = col * w
                    gt1 = b_ > m1
                    gt2 = b_ > m2
                    nm3 = jnp.maximum(m3, jnp.minimum(m2, b_))
                    nm2 = jnp.maximum(m2, jnp.minimum(m1, b_))
                    ni2 = jnp.where(gt1, i1, jnp.where(gt2, evk, i2))
                    nm1 = jnp.maximum(m1, b_)
                    ni1 = jnp.where(gt1, evk, i1)
                    m1, m2, m3, i1, i2 = nm1, nm2, nm3, ni1, ni2
                return m1, m2, m3, i1, i2, ev + UNROLL

            m1, m2, m3, i1, i2, _ = lax.fori_loop(
                0, NUM_EXPERTS // UNROLL, expert_body,
                (neg_inf, neg_inf, neg_inf, zeros, zeros, zeros))

            t = jnp.exp(m2 - m1)
            inv = 1.0 / (1.0 + t)
            off = chunk * CHUNK + g * LANES
            e0_v[pl.ds(off, LANES)] = i1
            e1_v[pl.ds(off, LANES)] = i2
            w0_v[pl.ds(off, LANES)] = inv
            w1_v[pl.ds(off, LANES)] = t * inv
            p_v[pl.ds(off, LANES)] = m3
            return carry

        lax.fori_loop(0, GPC, group_body, 0)

    pltpu.sync_copy(e0_v, oidx_hbm.at[0, b, pl.ds(row0, TPW)])
    pltpu.sync_copy(e1_v, oidx_hbm.at[1, b, pl.ds(row0, TPW)])
    pltpu.sync_copy(w0_v, orw_hbm.at[0, b, pl.ds(row0, TPW)])
    pltpu.sync_copy(w1_v, orw_hbm.at[1, b, pl.ds(row0, TPW)])
    pltpu.sync_copy(p_v, opay_hbm.at[0, b, pl.ds(row0, TPW)])
    pltpu.sync_copy(p_v, opay_hbm.at[1, b, pl.ds(row0, TPW)])


def kernel(confidences, wealth):
    oidx, orw, opay = _auction(confidences, wealth)
    perm = (1, 2, 0)
    return (jnp.transpose(oidx, perm), jnp.transpose(orw, perm),
            jnp.transpose(opay, perm))
